# baseline (speedup 1.0000x reference)
import functools
import jax
import jax.numpy as jnp
from jax import lax
from jax.experimental import pallas as pl
from jax.experimental.pallas import tpu as pltpu
from jax.experimental.pallas import tpu_sc as plsc

_N = 4096
_T = 1024
_NW = 16
_CHUNK = _N // _NW
_L = 16
_ROW = 2 * _T + 128
_EPS = 1e-07


def _sc_hist_body(h_hbm, ev_hbm, et_hbm, out_hbm, h_v, ev_v, et_v, hist_v, sem):
    wid = lax.axis_index("s")
    base = wid * _CHUNK
    cp1 = pltpu.async_copy(h_hbm.at[pl.ds(base, _CHUNK)], h_v, sem)
    cp2 = pltpu.async_copy(ev_hbm.at[pl.ds(base, _CHUNK)], ev_v, sem)
    cp3 = pltpu.async_copy(et_hbm.at[pl.ds(base, _CHUNK)], et_v, sem)
    zeros = jnp.zeros((_L,), jnp.float32)
    for i in range(_ROW // _L):
        hist_v[pl.ds(i * _L, _L)] = zeros
    cp1.wait()
    cp2.wait()
    cp3.wait()
    acc = zeros
    for k in range(_CHUNK // _L):
        sl = pl.ds(k * _L, _L)
        idx = et_v[sl]
        plsc.addupdate_scatter(hist_v, [idx], jnp.exp(h_v[sl]))
        plsc.addupdate_scatter(hist_v, [idx + _T], ev_v[sl])
        acc = acc + ev_v[sl] * h_v[sl]
    hist_v[pl.ds(2 * _T, _L)] = acc
    pltpu.sync_copy(hist_v, out_hbm.at[wid])


@functools.cache
def _sc_hist():
    return pl.kernel(
        _sc_hist_body,
        out_type=jax.ShapeDtypeStruct((_NW, _ROW), jnp.float32),
        mesh=plsc.VectorSubcoreMesh(core_axis_name="c", subcore_axis_name="s",
                                    num_cores=1, num_subcores=_NW),
        scratch_types=[
            pltpu.VMEM((_CHUNK,), jnp.float32),
            pltpu.VMEM((_CHUNK,), jnp.float32),
            pltpu.VMEM((_CHUNK,), jnp.int32),
            pltpu.VMEM((_ROW,), jnp.float32),
            pltpu.SemaphoreType.DMA,
        ],
        compiler_params=pltpu.CompilerParams(needs_layout_passes=False),
    )


def _tc_fin_body(hist_ref, out_ref):
    hs = jnp.sum(hist_ref[...], axis=0, keepdims=True)
    w = hs[:, :_T]
    c = hs[:, _T:2 * _T]
    evh = jnp.sum(hs[:, 2 * _T:])
    ra = lax.broadcasted_iota(jnp.int32, (_T, _T), 0)
    rb = lax.broadcasted_iota(jnp.int32, (_T, _T), 1)
    tri = (ra >= rb).astype(jnp.float32)
    suffix = jnp.dot(w, tri, preferred_element_type=jnp.float32,
                     precision=lax.Precision.HIGHEST)
    lterm = jnp.sum(jnp.where(c > 0., c * jnp.log(jnp.maximum(suffix, 1e-37)), 0.))
    nev = jnp.sum(c)
    out_ref[0, 0] = (lterm - evh) / (nev + _EPS)


def _tc_fin(hist):
    return pl.pallas_call(
        _tc_fin_body,
        out_specs=pl.BlockSpec(memory_space=pltpu.SMEM),
        out_shape=jax.ShapeDtypeStruct((1, 1), jnp.float32),
    )(hist)


def kernel(hazard, is_event, event_time):
    h = hazard.reshape(-1).astype(jnp.float32)
    ev = is_event.astype(jnp.float32).reshape(-1)
    eti = event_time.astype(jnp.int32)
    hist = _sc_hist()(h, ev, eti)
    out = _tc_fin(hist)
    return out[0, 0]
